# baseline (speedup 1.0000x reference)
import functools

import jax
import jax.numpy as jnp
from jax import lax
from jax.experimental import pallas as pl
from jax.experimental.pallas import tpu as pltpu
from jax.experimental.pallas import tpu_sc as plsc

RES = 256
N = RES ** 3
NROWS = N // 128
NQ = 262_144
SEARCH_LEVELS = 25


_CB = 2048


def _scan128_kernel(s_ref, xt_ref, yt_ref, last_ref, p_ref):
    r = s_ref[0, 0]
    p_ref[...] = xt_ref[...] * r
    acc = p_ref[0:1, :]
    yt_ref[0:1, :] = acc
    for i in range(1, 128):
        acc = acc + p_ref[i : i + 1, :]
        yt_ref[i : i + 1, :] = acc
    last_ref[...] = acc


def _scan128(xt, r, cb):
    m = xt.shape[1]
    grid = m // cb
    return pl.pallas_call(
        _scan128_kernel,
        grid=(grid,),
        in_specs=[
            pl.BlockSpec(memory_space=pltpu.SMEM),
            pl.BlockSpec((128, cb), lambda i: (0, i)),
        ],
        out_specs=[
            pl.BlockSpec((128, cb), lambda i: (0, i)),
            pl.BlockSpec((1, cb), lambda i: (0, i)),
        ],
        out_shape=[
            jax.ShapeDtypeStruct((128, m), jnp.float32),
            jax.ShapeDtypeStruct((1, m), jnp.float32),
        ],
        scratch_shapes=[pltpu.VMEM((128, cb), jnp.float32)],
    )(r.reshape(1, 1), xt)


def _scan128_noscale_kernel(xt_ref, yt_ref, last_ref):
    p = xt_ref[...]
    acc = p[0:1, :]
    yt_ref[0:1, :] = acc
    for i in range(1, 128):
        acc = acc + p[i : i + 1, :]
        yt_ref[i : i + 1, :] = acc
    last_ref[...] = acc


def _scan128_noscale(xt):
    m = xt.shape[1]
    return pl.pallas_call(
        _scan128_noscale_kernel,
        grid=(1,),
        in_specs=[pl.BlockSpec((128, m), lambda i: (0, 0))],
        out_specs=[
            pl.BlockSpec((128, m), lambda i: (0, 0)),
            pl.BlockSpec((1, m), lambda i: (0, 0)),
        ],
        out_shape=[
            jax.ShapeDtypeStruct((128, m), jnp.float32),
            jax.ShapeDtypeStruct((1, m), jnp.float32),
        ],
    )(xt)


def _level3_kernel(i3_ref, s3_ref, cs2_ref):
    s3 = s3_ref[...]
    off = jnp.zeros((1, 1), jnp.float32)
    cols = [off]
    for a in range(1, 8):
        off = off + s3[0:1, a - 1 : a]
        cols.append(off)
    off3 = jnp.concatenate(cols, axis=1)
    cs2_ref[...] = i3_ref[...] + off3


def _level3(inner3_t, s3):
    return pl.pallas_call(
        _level3_kernel,
        grid=(1,),
        in_specs=[
            pl.BlockSpec((128, 8), lambda i: (0, 0)),
            pl.BlockSpec((1, 8), lambda i: (0, 0)),
        ],
        out_specs=pl.BlockSpec((128, 8), lambda i: (0, 0)),
        out_shape=jax.ShapeDtypeStruct((128, 8), jnp.float32),
    )(inner3_t, s3)


def _add_bcast_kernel(x_ref, o_ref, y_ref):
    y_ref[...] = x_ref[...] + o_ref[...]


def _add_bcast(xt, off, cb):
    m = xt.shape[1]
    grid = m // cb
    return pl.pallas_call(
        _add_bcast_kernel,
        grid=(grid,),
        in_specs=[
            pl.BlockSpec((128, cb), lambda i: (0, i)),
            pl.BlockSpec((1, cb), lambda i: (0, i)),
        ],
        out_specs=pl.BlockSpec((128, cb), lambda i: (0, i)),
        out_shape=jax.ShapeDtypeStruct((128, m), jnp.float32),
    )(xt, off)


_QW = NQ // 32
_QROWS = _QW // 128
_TREE_LEVELS = 7
_TREE_SIZE = 1 << _TREE_LEVELS


def _tree_rowmap():
    import numpy as np

    m = np.zeros(_TREE_SIZE, np.int32)
    for lvl in range(_TREE_LEVELS):
        j = np.arange(1 << lvl)
        m[(1 << lvl) + j] = (2 * j + 1) * (1 << (6 - lvl))
    return m


def _search_body(cdf_hbm, tb_hbm, u_hbm, nh_hbm, nw_hbm, nd_hbm,
                 oh_hbm, ow_hbm, od_hbm,
                 u_v, lo_v, hi_v, mid_v, val_v, a_v, b_v, c_v,
                 tb_v, addr_v, sem):
    nc = 2
    wid = lax.axis_index("s") * nc + lax.axis_index("c")

    pltpu.sync_copy(u_hbm.at[wid], u_v)
    pltpu.sync_copy(tb_hbm, tb_v)

    def _sel(base_row, width, path, shift):
        if width == 1:
            return tb_v[pl.ds(base_row * 16, 16)]
        half = width // 2
        bit = jnp.bitwise_and(lax.shift_right_logical(path, shift), 1)
        lv = _sel(base_row, half, path, shift - 1)
        rv = _sel(base_row + half, half, path, shift - 1)
        return jnp.where(bit == 1, rv, lv)

    def tree_body(i, _):
        one = jnp.ones((16,), jnp.int32)
        zero = jnp.zeros((16,), jnp.int32)
        for t in range(4):
            sl = pl.ds((i * 4 + t) * 16, 16)
            u = u_v[sl]
            node = one
            for lvl in range(_TREE_LEVELS):
                path = node - (1 << lvl)
                v = _sel(1 << lvl, 1 << lvl, path, lvl - 1)
                node = node + node + jnp.where(u <= v, zero, one)
            lo = node * (1 << 17) - N
            lo_v[sl] = lo
            hi_v[sl] = lo + (1 << 17)
            mid = lo + (1 << 16)
            mid_v[sl] = mid
            addr_v[sl] = (jnp.bitwise_and(mid, 127) * (1 << 17)
                          + lax.shift_right_logical(mid, 7))
        return 0

    lax.fori_loop(0, _QW // 64, tree_body, 0)

    for _level in range(_TREE_LEVELS, SEARCH_LEVELS):
        def issue_body(b, _):
            sl = pl.ds(b * 128, 128)
            pltpu.async_copy(cdf_hbm.at[addr_v.at[sl]], val_v.at[sl], sem)
            return 0

        lax.fori_loop(0, _QW // 128, issue_body, 0)

        def drain_body(b, _):
            sl = pl.ds(b * 128, 128)
            pltpu.make_async_copy(cdf_hbm.at[addr_v.at[sl]], val_v.at[sl], sem).wait()
            return 0

        lax.fori_loop(0, _QW // 128, drain_body, 0)

        def upd_body(i, _):
            for t in range(4):
                sl = pl.ds((i * 4 + t) * 16, 16)
                le = u_v[sl] <= val_v[sl]
                mid = mid_v[sl]
                hi = jnp.where(le, mid, hi_v[sl])
                lo = jnp.where(le, lo_v[sl], mid)
                hi_v[sl] = hi
                lo_v[sl] = lo
                mid = lo + lax.shift_right_logical(hi - lo, 1)
                mid_v[sl] = mid
                addr_v[sl] = (jnp.bitwise_and(mid, 127) * (1 << 17)
                              + lax.shift_right_logical(mid, 7))
            return 0

        lax.fori_loop(0, _QW // 64, upd_body, 0)

    pltpu.sync_copy(nh_hbm.at[wid], a_v)
    pltpu.sync_copy(nw_hbm.at[wid], b_v)
    pltpu.sync_copy(nd_hbm.at[wid], c_v)

    inv = jnp.float32(1.0 / 256.0)

    def dec_body(i, _):
        sl = pl.ds(i * 16, 16)
        s = jnp.minimum(hi_v[sl], N - 1)
        h = lax.shift_right_logical(s, 16)
        w = jnp.bitwise_and(lax.shift_right_logical(s, 8), 255)
        d = jnp.bitwise_and(s, 255)
        a_v[sl] = (h.astype(jnp.float32) + a_v[sl]) * inv
        b_v[sl] = (w.astype(jnp.float32) + b_v[sl]) * inv
        c_v[sl] = (d.astype(jnp.float32) + c_v[sl]) * inv
        return 0

    lax.fori_loop(0, _QW // 16, dec_body, 0)

    pltpu.sync_copy(a_v, oh_hbm.at[wid])
    pltpu.sync_copy(b_v, ow_hbm.at[wid])
    pltpu.sync_copy(c_v, od_hbm.at[wid])


def _sc_search(cdf, tb, u, nh, nw, nd):
    mesh = plsc.VectorSubcoreMesh(core_axis_name="c", subcore_axis_name="s")
    f = pl.kernel(
        _search_body,
        out_type=[jax.ShapeDtypeStruct((32, _QW), jnp.float32)] * 3,
        mesh=mesh,
        scratch_types=[
            pltpu.VMEM((_QW,), jnp.float32),
            pltpu.VMEM((_QW,), jnp.int32),
            pltpu.VMEM((_QW,), jnp.int32),
            pltpu.VMEM((_QW,), jnp.int32),
            pltpu.VMEM((_QW,), jnp.float32),
            pltpu.VMEM((_QW,), jnp.float32),
            pltpu.VMEM((_QW,), jnp.float32),
            pltpu.VMEM((_QW,), jnp.float32),
            pltpu.VMEM((_TREE_SIZE * 16,), jnp.float32),
            pltpu.VMEM((_QW,), jnp.int32),
            pltpu.SemaphoreType.DMA,
        ],
    )
    shp = (32, _QW)
    return f(cdf, tb, u.reshape(shp), nh.reshape(shp), nw.reshape(shp),
             nd.reshape(shp))


def kernel(weights, num_points):
    flat = weights.reshape(-1)
    S = flat.sum()
    r = jnp.float32(1.0) / S

    xt = flat.reshape(NROWS, 128).T
    inner1_t, s1 = _scan128(xt, r, _CB)

    s1t = s1.reshape(NROWS // 128, 128).T
    inner2_t, s2 = _scan128_noscale(s1t)

    s2t = s2.reshape(8, 128).T
    inner3_t, s3 = _scan128_noscale(s2t)

    cs2_t = _level3(inner3_t, s3)
    cs2 = cs2_t.T.reshape(-1)
    off2 = jnp.concatenate([jnp.zeros((1,), jnp.float32), cs2[:-1]])

    cs1_t = _add_bcast(inner2_t, off2.reshape(1, -1), 1024)
    cs1 = cs1_t.T.reshape(-1)
    off1 = jnp.concatenate([jnp.zeros((1,), jnp.float32), cs1[:-1]])

    cdf_t = _add_bcast(inner1_t, off1.reshape(1, -1), _CB)
    cdf_flat_t = cdf_t.reshape(-1)

    key = jax.random.key(42)
    k_u, k_r = jax.random.split(key)
    u = jax.random.uniform(k_u, (NQ,), dtype=jnp.float32)
    noise = jax.random.uniform(k_r, (NQ, 3), dtype=jnp.float32)

    grid = lax.slice_in_dim(cdf_t, 0, 1, axis=0).reshape(-1)[::1024]
    table = grid[jnp.asarray(_tree_rowmap())]
    tb = jnp.broadcast_to(table[:, None], (_TREE_SIZE, 16)).reshape(-1)

    oh, ow, od = _sc_search(cdf_flat_t, tb, u,
                            noise[:, 0], noise[:, 1], noise[:, 2])

    return jnp.stack(
        [oh.reshape(-1), ow.reshape(-1), od.reshape(-1)], axis=1
    )


# speedup vs baseline: 1.0600x; 1.0600x over previous
import functools

import jax
import jax.numpy as jnp
from jax import lax
from jax.experimental import pallas as pl
from jax.experimental.pallas import tpu as pltpu
from jax.experimental.pallas import tpu_sc as plsc

RES = 256
N = RES ** 3
NROWS = N // 128
NQ = 262_144
SEARCH_LEVELS = 25


_CB = 4096


def _scan128_kernel(s_ref, xt_ref, yt_ref, last_ref, p_ref):
    r = s_ref[0, 0]
    p_ref[...] = xt_ref[...] * r
    acc = p_ref[0:1, :]
    yt_ref[0:1, :] = acc
    for i in range(1, 128):
        acc = acc + p_ref[i : i + 1, :]
        yt_ref[i : i + 1, :] = acc
    last_ref[...] = acc


def _scan128(xt, r, cb):
    m = xt.shape[1]
    grid = m // cb
    return pl.pallas_call(
        _scan128_kernel,
        grid=(grid,),
        in_specs=[
            pl.BlockSpec(memory_space=pltpu.SMEM),
            pl.BlockSpec((128, cb), lambda i: (0, i)),
        ],
        out_specs=[
            pl.BlockSpec((128, cb), lambda i: (0, i)),
            pl.BlockSpec((1, cb), lambda i: (0, i)),
        ],
        out_shape=[
            jax.ShapeDtypeStruct((128, m), jnp.float32),
            jax.ShapeDtypeStruct((1, m), jnp.float32),
        ],
        scratch_shapes=[pltpu.VMEM((128, cb), jnp.float32)],
    )(r.reshape(1, 1), xt)


def _scan128_noscale_kernel(xt_ref, yt_ref, last_ref):
    p = xt_ref[...]
    acc = p[0:1, :]
    yt_ref[0:1, :] = acc
    for i in range(1, 128):
        acc = acc + p[i : i + 1, :]
        yt_ref[i : i + 1, :] = acc
    last_ref[...] = acc


def _scan128_noscale(xt):
    m = xt.shape[1]
    return pl.pallas_call(
        _scan128_noscale_kernel,
        grid=(1,),
        in_specs=[pl.BlockSpec((128, m), lambda i: (0, 0))],
        out_specs=[
            pl.BlockSpec((128, m), lambda i: (0, 0)),
            pl.BlockSpec((1, m), lambda i: (0, 0)),
        ],
        out_shape=[
            jax.ShapeDtypeStruct((128, m), jnp.float32),
            jax.ShapeDtypeStruct((1, m), jnp.float32),
        ],
    )(xt)


def _level3_kernel(i3_ref, s3_ref, cs2_ref):
    s3 = s3_ref[...]
    off = jnp.zeros((1, 1), jnp.float32)
    cols = [off]
    for a in range(1, 8):
        off = off + s3[0:1, a - 1 : a]
        cols.append(off)
    off3 = jnp.concatenate(cols, axis=1)
    cs2_ref[...] = i3_ref[...] + off3


def _level3(inner3_t, s3):
    return pl.pallas_call(
        _level3_kernel,
        grid=(1,),
        in_specs=[
            pl.BlockSpec((128, 8), lambda i: (0, 0)),
            pl.BlockSpec((1, 8), lambda i: (0, 0)),
        ],
        out_specs=pl.BlockSpec((128, 8), lambda i: (0, 0)),
        out_shape=jax.ShapeDtypeStruct((128, 8), jnp.float32),
    )(inner3_t, s3)


def _add_bcast_kernel(x_ref, o_ref, y_ref):
    y_ref[...] = x_ref[...] + o_ref[...]


def _add_bcast(xt, off, cb):
    m = xt.shape[1]
    grid = m // cb
    return pl.pallas_call(
        _add_bcast_kernel,
        grid=(grid,),
        in_specs=[
            pl.BlockSpec((128, cb), lambda i: (0, i)),
            pl.BlockSpec((1, cb), lambda i: (0, i)),
        ],
        out_specs=pl.BlockSpec((128, cb), lambda i: (0, i)),
        out_shape=jax.ShapeDtypeStruct((128, m), jnp.float32),
    )(xt, off)


_QW = NQ // 32
_QROWS = _QW // 128
_TREE_LEVELS = 7
_TREE_SIZE = 1 << _TREE_LEVELS


def _tree_rowmap():
    import numpy as np

    m = np.zeros(_TREE_SIZE, np.int32)
    for lvl in range(_TREE_LEVELS):
        j = np.arange(1 << lvl)
        m[(1 << lvl) + j] = (2 * j + 1) * (1 << (6 - lvl))
    return m


def _search_body(cdf_hbm, tb_hbm, u_hbm, nh_hbm, nw_hbm, nd_hbm,
                 oh_hbm, ow_hbm, od_hbm,
                 u_v, lo_v, hi_v, mid_v, val_v, a_v, b_v, c_v,
                 tb_v, addr_v, sem):
    nc = 2
    wid = lax.axis_index("s") * nc + lax.axis_index("c")

    pltpu.sync_copy(u_hbm.at[wid], u_v)
    pltpu.sync_copy(tb_hbm, tb_v)

    def _sel(base_row, width, path, shift):
        if width == 1:
            return tb_v[pl.ds(base_row * 16, 16)]
        half = width // 2
        bit = jnp.bitwise_and(lax.shift_right_logical(path, shift), 1)
        lv = _sel(base_row, half, path, shift - 1)
        rv = _sel(base_row + half, half, path, shift - 1)
        return jnp.where(bit == 1, rv, lv)

    def tree_body(i, _):
        one = jnp.ones((16,), jnp.int32)
        zero = jnp.zeros((16,), jnp.int32)
        for t in range(4):
            sl = pl.ds((i * 4 + t) * 16, 16)
            u = u_v[sl]
            node = one
            for lvl in range(_TREE_LEVELS):
                path = node - (1 << lvl)
                v = _sel(1 << lvl, 1 << lvl, path, lvl - 1)
                node = node + node + jnp.where(u <= v, zero, one)
            lo = node * (1 << 17) - N
            lo_v[sl] = lo
            hi_v[sl] = lo + (1 << 17)
            mid = lo + (1 << 16)
            mid_v[sl] = mid
            addr_v[sl] = (jnp.bitwise_and(mid, 127) * (1 << 17)
                          + lax.shift_right_logical(mid, 7))
        return 0

    lax.fori_loop(0, _QW // 64, tree_body, 0)

    for _level in range(_TREE_LEVELS, SEARCH_LEVELS):
        def issue_body(b, _):
            sl = pl.ds(b * 256, 256)
            pltpu.async_copy(cdf_hbm.at[addr_v.at[sl]], val_v.at[sl], sem)
            return 0

        lax.fori_loop(0, _QW // 256, issue_body, 0)

        def drain_body(b, _):
            sl = pl.ds(b * 256, 256)
            pltpu.make_async_copy(cdf_hbm.at[addr_v.at[sl]], val_v.at[sl], sem).wait()
            return 0

        lax.fori_loop(0, _QW // 256, drain_body, 0)

        def upd_body(i, _):
            for t in range(4):
                sl = pl.ds((i * 4 + t) * 16, 16)
                le = u_v[sl] <= val_v[sl]
                mid = mid_v[sl]
                hi = jnp.where(le, mid, hi_v[sl])
                lo = jnp.where(le, lo_v[sl], mid)
                hi_v[sl] = hi
                lo_v[sl] = lo
                mid = lo + lax.shift_right_logical(hi - lo, 1)
                mid_v[sl] = mid
                addr_v[sl] = (jnp.bitwise_and(mid, 127) * (1 << 17)
                              + lax.shift_right_logical(mid, 7))
            return 0

        lax.fori_loop(0, _QW // 64, upd_body, 0)

    pltpu.sync_copy(nh_hbm.at[wid], a_v)
    pltpu.sync_copy(nw_hbm.at[wid], b_v)
    pltpu.sync_copy(nd_hbm.at[wid], c_v)

    inv = jnp.float32(1.0 / 256.0)

    def dec_body(i, _):
        sl = pl.ds(i * 16, 16)
        s = jnp.minimum(hi_v[sl], N - 1)
        h = lax.shift_right_logical(s, 16)
        w = jnp.bitwise_and(lax.shift_right_logical(s, 8), 255)
        d = jnp.bitwise_and(s, 255)
        a_v[sl] = (h.astype(jnp.float32) + a_v[sl]) * inv
        b_v[sl] = (w.astype(jnp.float32) + b_v[sl]) * inv
        c_v[sl] = (d.astype(jnp.float32) + c_v[sl]) * inv
        return 0

    lax.fori_loop(0, _QW // 16, dec_body, 0)

    pltpu.sync_copy(a_v, oh_hbm.at[wid])
    pltpu.sync_copy(b_v, ow_hbm.at[wid])
    pltpu.sync_copy(c_v, od_hbm.at[wid])


def _sc_search(cdf, tb, u, nh, nw, nd):
    mesh = plsc.VectorSubcoreMesh(core_axis_name="c", subcore_axis_name="s")
    f = pl.kernel(
        _search_body,
        out_type=[jax.ShapeDtypeStruct((32, _QW), jnp.float32)] * 3,
        mesh=mesh,
        scratch_types=[
            pltpu.VMEM((_QW,), jnp.float32),
            pltpu.VMEM((_QW,), jnp.int32),
            pltpu.VMEM((_QW,), jnp.int32),
            pltpu.VMEM((_QW,), jnp.int32),
            pltpu.VMEM((_QW,), jnp.float32),
            pltpu.VMEM((_QW,), jnp.float32),
            pltpu.VMEM((_QW,), jnp.float32),
            pltpu.VMEM((_QW,), jnp.float32),
            pltpu.VMEM((_TREE_SIZE * 16,), jnp.float32),
            pltpu.VMEM((_QW,), jnp.int32),
            pltpu.SemaphoreType.DMA,
        ],
    )
    shp = (32, _QW)
    return f(cdf, tb, u.reshape(shp), nh.reshape(shp), nw.reshape(shp),
             nd.reshape(shp))


def kernel(weights, num_points):
    flat = weights.reshape(-1)
    S = flat.sum()
    r = jnp.float32(1.0) / S

    xt = flat.reshape(NROWS, 128).T
    inner1_t, s1 = _scan128(xt, r, _CB)

    s1t = s1.reshape(NROWS // 128, 128).T
    inner2_t, s2 = _scan128_noscale(s1t)

    s2t = s2.reshape(8, 128).T
    inner3_t, s3 = _scan128_noscale(s2t)

    cs2_t = _level3(inner3_t, s3)
    cs2 = cs2_t.T.reshape(-1)
    off2 = jnp.concatenate([jnp.zeros((1,), jnp.float32), cs2[:-1]])

    cs1_t = _add_bcast(inner2_t, off2.reshape(1, -1), 1024)
    cs1 = cs1_t.T.reshape(-1)
    off1 = jnp.concatenate([jnp.zeros((1,), jnp.float32), cs1[:-1]])

    cdf_t = _add_bcast(inner1_t, off1.reshape(1, -1), _CB)
    cdf_flat_t = cdf_t.reshape(-1)

    key = jax.random.key(42)
    k_u, k_r = jax.random.split(key)
    u = jax.random.uniform(k_u, (NQ,), dtype=jnp.float32)
    noise = jax.random.uniform(k_r, (NQ, 3), dtype=jnp.float32)

    grid = lax.slice_in_dim(cdf_t, 0, 1, axis=0).reshape(-1)[::1024]
    table = grid[jnp.asarray(_tree_rowmap())]
    tb = jnp.broadcast_to(table[:, None], (_TREE_SIZE, 16)).reshape(-1)

    oh, ow, od = _sc_search(cdf_flat_t, tb, u,
                            noise[:, 0], noise[:, 1], noise[:, 2])

    return jnp.stack(
        [oh.reshape(-1), ow.reshape(-1), od.reshape(-1)], axis=1
    )


# speedup vs baseline: 1.0764x; 1.0155x over previous
import functools

import jax
import jax.numpy as jnp
from jax import lax
from jax.experimental import pallas as pl
from jax.experimental.pallas import tpu as pltpu
from jax.experimental.pallas import tpu_sc as plsc

RES = 256
N = RES ** 3
NROWS = N // 128
NQ = 262_144
SEARCH_LEVELS = 25


_CB = 8192


def _scan128_kernel(s_ref, xt_ref, yt_ref, last_ref, p_ref):
    r = s_ref[0, 0]
    p_ref[...] = xt_ref[...] * r
    acc = p_ref[0:1, :]
    yt_ref[0:1, :] = acc
    for i in range(1, 128):
        acc = acc + p_ref[i : i + 1, :]
        yt_ref[i : i + 1, :] = acc
    last_ref[...] = acc


def _scan128(xt, r, cb):
    m = xt.shape[1]
    grid = m // cb
    return pl.pallas_call(
        _scan128_kernel,
        grid=(grid,),
        in_specs=[
            pl.BlockSpec(memory_space=pltpu.SMEM),
            pl.BlockSpec((128, cb), lambda i: (0, i)),
        ],
        out_specs=[
            pl.BlockSpec((128, cb), lambda i: (0, i)),
            pl.BlockSpec((1, cb), lambda i: (0, i)),
        ],
        out_shape=[
            jax.ShapeDtypeStruct((128, m), jnp.float32),
            jax.ShapeDtypeStruct((1, m), jnp.float32),
        ],
        scratch_shapes=[pltpu.VMEM((128, cb), jnp.float32)],
    )(r.reshape(1, 1), xt)


def _scan128_noscale_kernel(xt_ref, yt_ref, last_ref):
    p = xt_ref[...]
    acc = p[0:1, :]
    yt_ref[0:1, :] = acc
    for i in range(1, 128):
        acc = acc + p[i : i + 1, :]
        yt_ref[i : i + 1, :] = acc
    last_ref[...] = acc


def _scan128_noscale(xt):
    m = xt.shape[1]
    return pl.pallas_call(
        _scan128_noscale_kernel,
        grid=(1,),
        in_specs=[pl.BlockSpec((128, m), lambda i: (0, 0))],
        out_specs=[
            pl.BlockSpec((128, m), lambda i: (0, 0)),
            pl.BlockSpec((1, m), lambda i: (0, 0)),
        ],
        out_shape=[
            jax.ShapeDtypeStruct((128, m), jnp.float32),
            jax.ShapeDtypeStruct((1, m), jnp.float32),
        ],
    )(xt)


def _level3_kernel(i3_ref, s3_ref, cs2_ref):
    s3 = s3_ref[...]
    off = jnp.zeros((1, 1), jnp.float32)
    cols = [off]
    for a in range(1, 8):
        off = off + s3[0:1, a - 1 : a]
        cols.append(off)
    off3 = jnp.concatenate(cols, axis=1)
    cs2_ref[...] = i3_ref[...] + off3


def _level3(inner3_t, s3):
    return pl.pallas_call(
        _level3_kernel,
        grid=(1,),
        in_specs=[
            pl.BlockSpec((128, 8), lambda i: (0, 0)),
            pl.BlockSpec((1, 8), lambda i: (0, 0)),
        ],
        out_specs=pl.BlockSpec((128, 8), lambda i: (0, 0)),
        out_shape=jax.ShapeDtypeStruct((128, 8), jnp.float32),
    )(inner3_t, s3)


def _add_bcast_kernel(x_ref, o_ref, y_ref):
    y_ref[...] = x_ref[...] + o_ref[...]


def _add_bcast(xt, off, cb):
    m = xt.shape[1]
    grid = m // cb
    return pl.pallas_call(
        _add_bcast_kernel,
        grid=(grid,),
        in_specs=[
            pl.BlockSpec((128, cb), lambda i: (0, i)),
            pl.BlockSpec((1, cb), lambda i: (0, i)),
        ],
        out_specs=pl.BlockSpec((128, cb), lambda i: (0, i)),
        out_shape=jax.ShapeDtypeStruct((128, m), jnp.float32),
    )(xt, off)


_QW = NQ // 32
_QROWS = _QW // 128
_TREE_LEVELS = 7
_TREE_SIZE = 1 << _TREE_LEVELS


def _tree_rowmap():
    import numpy as np

    m = np.zeros(_TREE_SIZE, np.int32)
    for lvl in range(_TREE_LEVELS):
        j = np.arange(1 << lvl)
        m[(1 << lvl) + j] = (2 * j + 1) * (1 << (6 - lvl))
    return m


def _search_body(cdf_hbm, tb_hbm, u_hbm, nh_hbm, nw_hbm, nd_hbm,
                 oh_hbm, ow_hbm, od_hbm,
                 u_v, lo_v, hi_v, mid_v, val_v, a_v, b_v, c_v,
                 tb_v, addr_v, sem):
    nc = 2
    wid = lax.axis_index("s") * nc + lax.axis_index("c")

    pltpu.sync_copy(u_hbm.at[wid], u_v)
    pltpu.sync_copy(tb_hbm, tb_v)

    def _sel(base_row, width, path, shift):
        if width == 1:
            return tb_v[pl.ds(base_row * 16, 16)]
        half = width // 2
        bit = jnp.bitwise_and(lax.shift_right_logical(path, shift), 1)
        lv = _sel(base_row, half, path, shift - 1)
        rv = _sel(base_row + half, half, path, shift - 1)
        return jnp.where(bit == 1, rv, lv)

    def tree_body(i, _):
        one = jnp.ones((16,), jnp.int32)
        zero = jnp.zeros((16,), jnp.int32)
        for t in range(4):
            sl = pl.ds((i * 4 + t) * 16, 16)
            u = u_v[sl]
            node = one
            for lvl in range(_TREE_LEVELS):
                path = node - (1 << lvl)
                v = _sel(1 << lvl, 1 << lvl, path, lvl - 1)
                node = node + node + jnp.where(u <= v, zero, one)
            lo = node * (1 << 17) - N
            lo_v[sl] = lo
            hi_v[sl] = lo + (1 << 17)
            mid = lo + (1 << 16)
            mid_v[sl] = mid
            addr_v[sl] = (jnp.bitwise_and(mid, 127) * (1 << 17)
                          + lax.shift_right_logical(mid, 7))
        return 0

    lax.fori_loop(0, _QW // 64, tree_body, 0)

    for _level in range(_TREE_LEVELS, SEARCH_LEVELS):
        def issue_body(b, _):
            sl = pl.ds(b * 512, 512)
            pltpu.async_copy(cdf_hbm.at[addr_v.at[sl]], val_v.at[sl], sem)
            return 0

        lax.fori_loop(0, _QW // 512, issue_body, 0)

        def drain_body(b, _):
            sl = pl.ds(b * 512, 512)
            pltpu.make_async_copy(cdf_hbm.at[addr_v.at[sl]], val_v.at[sl], sem).wait()
            return 0

        lax.fori_loop(0, _QW // 512, drain_body, 0)

        def upd_body(i, _):
            for t in range(4):
                sl = pl.ds((i * 4 + t) * 16, 16)
                le = u_v[sl] <= val_v[sl]
                mid = mid_v[sl]
                hi = jnp.where(le, mid, hi_v[sl])
                lo = jnp.where(le, lo_v[sl], mid)
                hi_v[sl] = hi
                lo_v[sl] = lo
                mid = lo + lax.shift_right_logical(hi - lo, 1)
                mid_v[sl] = mid
                addr_v[sl] = (jnp.bitwise_and(mid, 127) * (1 << 17)
                              + lax.shift_right_logical(mid, 7))
            return 0

        lax.fori_loop(0, _QW // 64, upd_body, 0)

    pltpu.sync_copy(nh_hbm.at[wid], a_v)
    pltpu.sync_copy(nw_hbm.at[wid], b_v)
    pltpu.sync_copy(nd_hbm.at[wid], c_v)

    inv = jnp.float32(1.0 / 256.0)

    def dec_body(i, _):
        sl = pl.ds(i * 16, 16)
        s = jnp.minimum(hi_v[sl], N - 1)
        h = lax.shift_right_logical(s, 16)
        w = jnp.bitwise_and(lax.shift_right_logical(s, 8), 255)
        d = jnp.bitwise_and(s, 255)
        a_v[sl] = (h.astype(jnp.float32) + a_v[sl]) * inv
        b_v[sl] = (w.astype(jnp.float32) + b_v[sl]) * inv
        c_v[sl] = (d.astype(jnp.float32) + c_v[sl]) * inv
        return 0

    lax.fori_loop(0, _QW // 16, dec_body, 0)

    pltpu.sync_copy(a_v, oh_hbm.at[wid])
    pltpu.sync_copy(b_v, ow_hbm.at[wid])
    pltpu.sync_copy(c_v, od_hbm.at[wid])


def _sc_search(cdf, tb, u, nh, nw, nd):
    mesh = plsc.VectorSubcoreMesh(core_axis_name="c", subcore_axis_name="s")
    f = pl.kernel(
        _search_body,
        out_type=[jax.ShapeDtypeStruct((32, _QW), jnp.float32)] * 3,
        mesh=mesh,
        scratch_types=[
            pltpu.VMEM((_QW,), jnp.float32),
            pltpu.VMEM((_QW,), jnp.int32),
            pltpu.VMEM((_QW,), jnp.int32),
            pltpu.VMEM((_QW,), jnp.int32),
            pltpu.VMEM((_QW,), jnp.float32),
            pltpu.VMEM((_QW,), jnp.float32),
            pltpu.VMEM((_QW,), jnp.float32),
            pltpu.VMEM((_QW,), jnp.float32),
            pltpu.VMEM((_TREE_SIZE * 16,), jnp.float32),
            pltpu.VMEM((_QW,), jnp.int32),
            pltpu.SemaphoreType.DMA,
        ],
    )
    shp = (32, _QW)
    return f(cdf, tb, u.reshape(shp), nh.reshape(shp), nw.reshape(shp),
             nd.reshape(shp))


def kernel(weights, num_points):
    flat = weights.reshape(-1)
    S = flat.sum()
    r = jnp.float32(1.0) / S

    xt = flat.reshape(NROWS, 128).T
    inner1_t, s1 = _scan128(xt, r, _CB)

    s1t = s1.reshape(NROWS // 128, 128).T
    inner2_t, s2 = _scan128_noscale(s1t)

    s2t = s2.reshape(8, 128).T
    inner3_t, s3 = _scan128_noscale(s2t)

    cs2_t = _level3(inner3_t, s3)
    cs2 = cs2_t.T.reshape(-1)
    off2 = jnp.concatenate([jnp.zeros((1,), jnp.float32), cs2[:-1]])

    cs1_t = _add_bcast(inner2_t, off2.reshape(1, -1), 1024)
    cs1 = cs1_t.T.reshape(-1)
    off1 = jnp.concatenate([jnp.zeros((1,), jnp.float32), cs1[:-1]])

    cdf_t = _add_bcast(inner1_t, off1.reshape(1, -1), _CB)
    cdf_flat_t = cdf_t.reshape(-1)

    key = jax.random.key(42)
    k_u, k_r = jax.random.split(key)
    u = jax.random.uniform(k_u, (NQ,), dtype=jnp.float32)
    noise = jax.random.uniform(k_r, (NQ, 3), dtype=jnp.float32)

    grid = lax.slice_in_dim(cdf_t, 0, 1, axis=0).reshape(-1)[::1024]
    table = grid[jnp.asarray(_tree_rowmap())]
    tb = jnp.broadcast_to(table[:, None], (_TREE_SIZE, 16)).reshape(-1)

    oh, ow, od = _sc_search(cdf_flat_t, tb, u,
                            noise[:, 0], noise[:, 1], noise[:, 2])

    return jnp.stack(
        [oh.reshape(-1), ow.reshape(-1), od.reshape(-1)], axis=1
    )


# speedup vs baseline: 1.1120x; 1.0330x over previous
import functools

import jax
import jax.numpy as jnp
from jax import lax
from jax.experimental import pallas as pl
from jax.experimental.pallas import tpu as pltpu
from jax.experimental.pallas import tpu_sc as plsc

RES = 256
N = RES ** 3
NROWS = N // 128
NQ = 262_144
SEARCH_LEVELS = 25


_CB = 8192


def _scan128_kernel(s_ref, xt_ref, yt_ref, last_ref, p_ref):
    r = s_ref[0, 0]
    p_ref[...] = xt_ref[...] * r
    acc = p_ref[0:1, :]
    yt_ref[0:1, :] = acc
    for i in range(1, 128):
        acc = acc + p_ref[i : i + 1, :]
        yt_ref[i : i + 1, :] = acc
    last_ref[...] = acc


def _scan128(xt, r, cb):
    m = xt.shape[1]
    grid = m // cb
    return pl.pallas_call(
        _scan128_kernel,
        grid=(grid,),
        in_specs=[
            pl.BlockSpec(memory_space=pltpu.SMEM),
            pl.BlockSpec((128, cb), lambda i: (0, i)),
        ],
        out_specs=[
            pl.BlockSpec((128, cb), lambda i: (0, i)),
            pl.BlockSpec((1, cb), lambda i: (0, i)),
        ],
        out_shape=[
            jax.ShapeDtypeStruct((128, m), jnp.float32),
            jax.ShapeDtypeStruct((1, m), jnp.float32),
        ],
        scratch_shapes=[pltpu.VMEM((128, cb), jnp.float32)],
    )(r.reshape(1, 1), xt)


def _scan128_noscale_kernel(xt_ref, yt_ref, last_ref):
    p = xt_ref[...]
    acc = p[0:1, :]
    yt_ref[0:1, :] = acc
    for i in range(1, 128):
        acc = acc + p[i : i + 1, :]
        yt_ref[i : i + 1, :] = acc
    last_ref[...] = acc


def _scan128_noscale(xt):
    m = xt.shape[1]
    return pl.pallas_call(
        _scan128_noscale_kernel,
        grid=(1,),
        in_specs=[pl.BlockSpec((128, m), lambda i: (0, 0))],
        out_specs=[
            pl.BlockSpec((128, m), lambda i: (0, 0)),
            pl.BlockSpec((1, m), lambda i: (0, 0)),
        ],
        out_shape=[
            jax.ShapeDtypeStruct((128, m), jnp.float32),
            jax.ShapeDtypeStruct((1, m), jnp.float32),
        ],
    )(xt)


def _level3_kernel(i3_ref, s3_ref, cs2_ref):
    s3 = s3_ref[...]
    off = jnp.zeros((1, 1), jnp.float32)
    cols = [off]
    for a in range(1, 8):
        off = off + s3[0:1, a - 1 : a]
        cols.append(off)
    off3 = jnp.concatenate(cols, axis=1)
    cs2_ref[...] = i3_ref[...] + off3


def _level3(inner3_t, s3):
    return pl.pallas_call(
        _level3_kernel,
        grid=(1,),
        in_specs=[
            pl.BlockSpec((128, 8), lambda i: (0, 0)),
            pl.BlockSpec((1, 8), lambda i: (0, 0)),
        ],
        out_specs=pl.BlockSpec((128, 8), lambda i: (0, 0)),
        out_shape=jax.ShapeDtypeStruct((128, 8), jnp.float32),
    )(inner3_t, s3)


def _add_bcast_kernel(x_ref, o_ref, y_ref):
    y_ref[...] = x_ref[...] + o_ref[...]


def _add_bcast(xt, off, cb):
    m = xt.shape[1]
    grid = m // cb
    return pl.pallas_call(
        _add_bcast_kernel,
        grid=(grid,),
        in_specs=[
            pl.BlockSpec((128, cb), lambda i: (0, i)),
            pl.BlockSpec((1, cb), lambda i: (0, i)),
        ],
        out_specs=pl.BlockSpec((128, cb), lambda i: (0, i)),
        out_shape=jax.ShapeDtypeStruct((128, m), jnp.float32),
    )(xt, off)


_QW = NQ // 32
_QROWS = _QW // 128
_TREE_LEVELS = 7
_TREE_SIZE = 1 << _TREE_LEVELS


def _tree_rowmap():
    import numpy as np

    m = np.zeros(_TREE_SIZE, np.int32)
    for lvl in range(_TREE_LEVELS):
        j = np.arange(1 << lvl)
        m[(1 << lvl) + j] = (2 * j + 1) * (1 << (6 - lvl))
    return m


def _search_body(cdf_hbm, tb_hbm, u_hbm, nh_hbm, nw_hbm, nd_hbm,
                 oh_hbm, ow_hbm, od_hbm,
                 u_v, lo_v, hi_v, mid_v, val_v, a_v, b_v, c_v,
                 tb_v, addr_v, sem, sem2):
    nc = 2
    wid = lax.axis_index("s") * nc + lax.axis_index("c")

    pltpu.sync_copy(u_hbm.at[wid], u_v)
    pltpu.sync_copy(tb_hbm, tb_v)

    def _sel(base_row, width, path, shift):
        if width == 1:
            return tb_v[pl.ds(base_row * 16, 16)]
        half = width // 2
        bit = jnp.bitwise_and(lax.shift_right_logical(path, shift), 1)
        lv = _sel(base_row, half, path, shift - 1)
        rv = _sel(base_row + half, half, path, shift - 1)
        return jnp.where(bit == 1, rv, lv)

    def tree_body(i, _):
        one = jnp.ones((16,), jnp.int32)
        zero = jnp.zeros((16,), jnp.int32)
        for t in range(4):
            sl = pl.ds((i * 4 + t) * 16, 16)
            u = u_v[sl]
            node = one
            for lvl in range(_TREE_LEVELS):
                path = node - (1 << lvl)
                v = _sel(1 << lvl, 1 << lvl, path, lvl - 1)
                node = node + node + jnp.where(u <= v, zero, one)
            lo = node * (1 << 17) - N
            lo_v[sl] = lo
            hi_v[sl] = lo + (1 << 17)
            mid = lo + (1 << 16)
            mid_v[sl] = mid
            addr_v[sl] = (jnp.bitwise_and(mid, 127) * (1 << 17)
                          + lax.shift_right_logical(mid, 7))
        return 0

    lax.fori_loop(0, _QW // 64, tree_body, 0)

    _HC = _QW // 1024

    def _issue(half, s):
        def issue_body(b, _):
            sl = pl.ds((half * _HC + b) * 512, 512)
            pltpu.async_copy(cdf_hbm.at[addr_v.at[sl]], val_v.at[sl], s)
            return 0

        lax.fori_loop(0, _HC, issue_body, 0)

    def _drain(half, s):
        def drain_body(b, _):
            sl = pl.ds((half * _HC + b) * 512, 512)
            pltpu.make_async_copy(cdf_hbm.at[addr_v.at[sl]], val_v.at[sl], s).wait()
            return 0

        lax.fori_loop(0, _HC, drain_body, 0)

    def _update(half):
        def upd_body(i, _):
            for t in range(4):
                sl = pl.ds((i * 4 + t) * 16, 16)
                le = u_v[sl] <= val_v[sl]
                mid = mid_v[sl]
                hi = jnp.where(le, mid, hi_v[sl])
                lo = jnp.where(le, lo_v[sl], mid)
                hi_v[sl] = hi
                lo_v[sl] = lo
                mid = lo + lax.shift_right_logical(hi - lo, 1)
                mid_v[sl] = mid
                addr_v[sl] = (jnp.bitwise_and(mid, 127) * (1 << 17)
                              + lax.shift_right_logical(mid, 7))
            return 0

        base = half * (_QW // 128)
        lax.fori_loop(base, base + _QW // 128, upd_body, 0)

    _issue(0, sem)
    _issue(1, sem2)
    for _level in range(_TREE_LEVELS, SEARCH_LEVELS):
        last = _level == SEARCH_LEVELS - 1
        _drain(0, sem)
        _update(0)
        if not last:
            _issue(0, sem)
        _drain(1, sem2)
        _update(1)
        if not last:
            _issue(1, sem2)

    pltpu.sync_copy(nh_hbm.at[wid], a_v)
    pltpu.sync_copy(nw_hbm.at[wid], b_v)
    pltpu.sync_copy(nd_hbm.at[wid], c_v)

    inv = jnp.float32(1.0 / 256.0)

    def dec_body(i, _):
        sl = pl.ds(i * 16, 16)
        s = jnp.minimum(hi_v[sl], N - 1)
        h = lax.shift_right_logical(s, 16)
        w = jnp.bitwise_and(lax.shift_right_logical(s, 8), 255)
        d = jnp.bitwise_and(s, 255)
        a_v[sl] = (h.astype(jnp.float32) + a_v[sl]) * inv
        b_v[sl] = (w.astype(jnp.float32) + b_v[sl]) * inv
        c_v[sl] = (d.astype(jnp.float32) + c_v[sl]) * inv
        return 0

    lax.fori_loop(0, _QW // 16, dec_body, 0)

    pltpu.sync_copy(a_v, oh_hbm.at[wid])
    pltpu.sync_copy(b_v, ow_hbm.at[wid])
    pltpu.sync_copy(c_v, od_hbm.at[wid])


def _sc_search(cdf, tb, u, nh, nw, nd):
    mesh = plsc.VectorSubcoreMesh(core_axis_name="c", subcore_axis_name="s")
    f = pl.kernel(
        _search_body,
        out_type=[jax.ShapeDtypeStruct((32, _QW), jnp.float32)] * 3,
        mesh=mesh,
        scratch_types=[
            pltpu.VMEM((_QW,), jnp.float32),
            pltpu.VMEM((_QW,), jnp.int32),
            pltpu.VMEM((_QW,), jnp.int32),
            pltpu.VMEM((_QW,), jnp.int32),
            pltpu.VMEM((_QW,), jnp.float32),
            pltpu.VMEM((_QW,), jnp.float32),
            pltpu.VMEM((_QW,), jnp.float32),
            pltpu.VMEM((_QW,), jnp.float32),
            pltpu.VMEM((_TREE_SIZE * 16,), jnp.float32),
            pltpu.VMEM((_QW,), jnp.int32),
            pltpu.SemaphoreType.DMA,
            pltpu.SemaphoreType.DMA,
        ],
    )
    shp = (32, _QW)
    return f(cdf, tb, u.reshape(shp), nh.reshape(shp), nw.reshape(shp),
             nd.reshape(shp))


def kernel(weights, num_points):
    flat = weights.reshape(-1)
    S = flat.sum()
    r = jnp.float32(1.0) / S

    xt = flat.reshape(NROWS, 128).T
    inner1_t, s1 = _scan128(xt, r, _CB)

    s1t = s1.reshape(NROWS // 128, 128).T
    inner2_t, s2 = _scan128_noscale(s1t)

    s2t = s2.reshape(8, 128).T
    inner3_t, s3 = _scan128_noscale(s2t)

    cs2_t = _level3(inner3_t, s3)
    cs2 = cs2_t.T.reshape(-1)
    off2 = jnp.concatenate([jnp.zeros((1,), jnp.float32), cs2[:-1]])

    cs1_t = _add_bcast(inner2_t, off2.reshape(1, -1), 1024)
    cs1 = cs1_t.T.reshape(-1)
    off1 = jnp.concatenate([jnp.zeros((1,), jnp.float32), cs1[:-1]])

    cdf_t = _add_bcast(inner1_t, off1.reshape(1, -1), _CB)
    cdf_flat_t = cdf_t.reshape(-1)

    key = jax.random.key(42)
    k_u, k_r = jax.random.split(key)
    u = jax.random.uniform(k_u, (NQ,), dtype=jnp.float32)
    noise = jax.random.uniform(k_r, (NQ, 3), dtype=jnp.float32)

    grid = lax.slice_in_dim(cdf_t, 0, 1, axis=0).reshape(-1)[::1024]
    table = grid[jnp.asarray(_tree_rowmap())]
    tb = jnp.broadcast_to(table[:, None], (_TREE_SIZE, 16)).reshape(-1)

    oh, ow, od = _sc_search(cdf_flat_t, tb, u,
                            noise[:, 0], noise[:, 1], noise[:, 2])

    return jnp.stack(
        [oh.reshape(-1), ow.reshape(-1), od.reshape(-1)], axis=1
    )
